# baseline (speedup 1.0000x reference)
import functools

import jax
import jax.numpy as jnp
from jax.experimental import pallas as pl
from jax.experimental.pallas import tpu as pltpu

BLK = 512
KRES = 3
RHS = 48

_MM = (((1,), (1,)), ((), ()))


def _proj_body(x_ref, wint_ref, bint_ref, h0_ref, hx0_ref):
    n = x_ref.shape[1]
    for b in range(x_ref.shape[0]):
        h0 = jnp.tanh(
            jax.lax.dot_general(wint_ref[...], x_ref[b], _MM,
                                preferred_element_type=jnp.float32)
            + bint_ref[...])
        h0_ref[b] = h0
        hx0_ref[b] = jnp.concatenate(
            [h0.astype(jnp.bfloat16),
             jnp.ones((1, n), jnp.bfloat16),
             jnp.zeros((RHS - h0.shape[0] - 1, n), jnp.bfloat16)], axis=0)


def _main_body(adj_ref, h0_ref, hx0_ref,
               wm1at_ref, wm1bt_ref, bm1t_ref, wm2t_ref, bm2t_ref,
               wm3t_ref, bm3t_ref, h3_ref, abf_scr, h_scr, hx_scr, sem1, sem2):
    t = pl.program_id(0)
    b = pl.program_id(1)
    i = pl.program_id(2)
    d_h = h0_ref.shape[1]
    cols = pl.ds(i * BLK, BLK)

    src = 1 - t % 2
    dst = t % 2

    @pl.when(jnp.logical_and(t == 0, jnp.logical_and(b == 0, i == 0)))
    def _():
        cp1 = pltpu.make_async_copy(h0_ref, h_scr.at[1], sem1)
        cp2 = pltpu.make_async_copy(hx0_ref, hx_scr.at[1], sem2)
        cp1.start()
        cp2.start()
        cp1.wait()
        cp2.wait()

    slot = jnp.minimum(i, KRES)
    srows = pl.ds(slot * BLK, BLK)

    @pl.when(jnp.logical_or(t == 0, i >= KRES))
    def _():
        abf_scr[b, srows] = adj_ref[0].astype(jnp.bfloat16)

    a16 = abf_scr[b, srows]
    hx = hx_scr[src, b]
    me = jax.lax.dot_general(hx, a16, _MM,
                             preferred_element_type=jnp.float32)
    dinv = 1.0 / jnp.maximum(me[d_h:d_h + 1, :], 1.0)
    m = me[:d_h, :] * dinv

    h_blk = h_scr[src, b, :, cols]

    u = jnp.dot(wm1at_ref[...], h_blk, preferred_element_type=jnp.float32)
    u = u + jnp.dot(wm1bt_ref[...], m, preferred_element_type=jnp.float32)
    u = jax.nn.relu(u + bm1t_ref[...])
    u = jax.nn.relu(
        jnp.dot(wm2t_ref[...], u, preferred_element_type=jnp.float32)
        + bm2t_ref[...])
    u = jnp.dot(wm3t_ref[...], u, preferred_element_type=jnp.float32) + bm3t_ref[...]
    h_new = h_blk + u

    @pl.when(t < 2)
    def _():
        h_scr[dst, b, :, cols] = h_new
        hx_scr[dst, b, :, cols] = jnp.concatenate(
            [h_new.astype(jnp.bfloat16),
             jnp.ones((1, BLK), jnp.bfloat16),
             jnp.zeros((RHS - d_h - 1, BLK), jnp.bfloat16)], axis=0)

    h3_ref[0] = h_new


def _readout_body(h3_ref, wr1mt_ref, br1mt_ref, wr2mt_ref, br2mt_ref,
                  wr1vt_ref, br1vt_ref, wr2vt_ref, br2vt_ref,
                  zm_ref, zlv_ref):
    n = h3_ref.shape[2]
    for b in range(h3_ref.shape[0]):
        pool = (jnp.sum(h3_ref[b], axis=1, keepdims=True) * (1.0 / n))
        hm = jax.nn.relu(
            jnp.dot(wr1mt_ref[...], pool, preferred_element_type=jnp.float32)
            + br1mt_ref[...])
        zm = jnp.dot(wr2mt_ref[...], hm, preferred_element_type=jnp.float32) + br2mt_ref[...]
        zm_ref[b] = zm.reshape(-1)
        hv = jax.nn.relu(
            jnp.dot(wr1vt_ref[...], pool, preferred_element_type=jnp.float32)
            + br1vt_ref[...])
        zlv = jnp.dot(wr2vt_ref[...], hv, preferred_element_type=jnp.float32) + br2vt_ref[...]
        zlv_ref[b] = zlv.reshape(-1)


def kernel(x, adj, W_in, b_in, Wm1, bm1, Wm2, bm2, Wm3, bm3,
           Wr1m, br1m, Wr2m, br2m, Wr1v, br1v, Wr2v, br2v):
    B, N, D_IN = x.shape
    D_H = W_in.shape[1]
    D_Z = Wr2m.shape[1]
    nb = N // BLK

    WinT = W_in.T
    Wm1aT, Wm1bT = Wm1[:D_H].T, Wm1[D_H:].T
    Wm2T, Wm3T = Wm2.T, Wm3.T
    Wr1mT, Wr2mT, Wr1vT, Wr2vT = Wr1m.T, Wr2m.T, Wr1v.T, Wr2v.T
    col = lambda v: v.reshape(-1, 1)
    binT, bm1T, bm2T, bm3T = col(b_in), col(bm1), col(bm2), col(bm3)
    br1mT, br2mT, br1vT, br2vT = col(br1m), col(br2m), col(br1v), col(br2v)

    f32 = jnp.float32
    bf16 = jnp.bfloat16

    h0, hx0 = pl.pallas_call(
        _proj_body,
        out_shape=[jax.ShapeDtypeStruct((B, D_H, N), f32),
                   jax.ShapeDtypeStruct((B, RHS, N), bf16)],
    )(x, WinT, binT)

    def full(arr):
        return pl.BlockSpec(arr.shape, lambda t, b, i: (0,) * arr.ndim)

    def h3_idx(t, b, i):
        return (b, 0, i)

    weights = (Wm1aT, Wm1bT, bm1T, Wm2T, bm2T, Wm3T, bm3T)

    h3 = pl.pallas_call(
        _main_body,
        grid=(3, B, nb),
        in_specs=[pl.BlockSpec(
                      (1, BLK, N),
                      lambda t, b, i: (b, jnp.where(t == 0, i,
                                                    jnp.maximum(i, KRES)), 0)),
                  full(h0), full(hx0)] + [full(w) for w in weights],
        out_specs=pl.BlockSpec((1, D_H, BLK), h3_idx),
        out_shape=jax.ShapeDtypeStruct((B, D_H, N), f32),
        scratch_shapes=[
            pltpu.VMEM((B, (KRES + 1) * BLK, N), bf16),
            pltpu.VMEM((2, B, D_H, N), f32),
            pltpu.VMEM((2, B, RHS, N), bf16),
            pltpu.SemaphoreType.DMA,
            pltpu.SemaphoreType.DMA,
        ],
        compiler_params=pltpu.CompilerParams(
            dimension_semantics=("arbitrary", "arbitrary", "arbitrary")),
    )(adj, h0, hx0, *weights)

    zm, zlv = pl.pallas_call(
        _readout_body,
        out_shape=[jax.ShapeDtypeStruct((B, D_Z), f32),
                   jax.ShapeDtypeStruct((B, D_Z), f32)],
    )(h3, Wr1mT, br1mT, Wr2mT, br2mT, Wr1vT, br1vT, Wr2vT, br2vT)
    return (zm, zlv)


# speedup vs baseline: 1.0667x; 1.0667x over previous
import functools

import jax
import jax.numpy as jnp
from jax.experimental import pallas as pl
from jax.experimental.pallas import tpu as pltpu

BLK = 1024
RHS = 48

_MM = (((1,), (1,)), ((), ()))


def _proj_body(x_ref, wint_ref, bint_ref, h0_ref, hx0_ref):
    n = x_ref.shape[1]
    for b in range(x_ref.shape[0]):
        h0 = jnp.tanh(
            jax.lax.dot_general(wint_ref[...], x_ref[b], _MM,
                                preferred_element_type=jnp.float32)
            + bint_ref[...])
        h0_ref[b] = h0
        hx0_ref[b] = jnp.concatenate(
            [h0.astype(jnp.bfloat16),
             jnp.ones((1, n), jnp.bfloat16),
             jnp.zeros((RHS - h0.shape[0] - 1, n), jnp.bfloat16)], axis=0)


def _main_body(adj_ref, h0_ref, hx0_ref,
               wm1at_ref, wm1bt_ref, bm1t_ref, wm2t_ref, bm2t_ref,
               wm3t_ref, bm3t_ref, h3_ref, h_scr, hx_scr, sem1, sem2):
    t = pl.program_id(0)
    b = pl.program_id(1)
    i = pl.program_id(2)
    d_h = h0_ref.shape[1]
    cols = pl.ds(i * BLK, BLK)

    src = 1 - t % 2
    dst = t % 2

    @pl.when(jnp.logical_and(t == 0, jnp.logical_and(b == 0, i == 0)))
    def _():
        cp1 = pltpu.make_async_copy(h0_ref, h_scr.at[1], sem1)
        cp2 = pltpu.make_async_copy(hx0_ref, hx_scr.at[1], sem2)
        cp1.start()
        cp2.start()
        cp1.wait()
        cp2.wait()

    a16 = adj_ref[0].astype(jnp.bfloat16)
    hx = hx_scr[src, b]
    me = jax.lax.dot_general(hx, a16, _MM,
                             preferred_element_type=jnp.float32)
    dinv = 1.0 / jnp.maximum(me[d_h:d_h + 1, :], 1.0)
    m = me[:d_h, :] * dinv

    h_blk = h_scr[src, b, :, cols]

    u = jnp.dot(wm1at_ref[...], h_blk, preferred_element_type=jnp.float32)
    u = u + jnp.dot(wm1bt_ref[...], m, preferred_element_type=jnp.float32)
    u = jax.nn.relu(u + bm1t_ref[...])
    u = jax.nn.relu(
        jnp.dot(wm2t_ref[...], u, preferred_element_type=jnp.float32)
        + bm2t_ref[...])
    u = jnp.dot(wm3t_ref[...], u, preferred_element_type=jnp.float32) + bm3t_ref[...]
    h_new = h_blk + u

    @pl.when(t < 2)
    def _():
        h_scr[dst, b, :, cols] = h_new
        hx_scr[dst, b, :, cols] = jnp.concatenate(
            [h_new.astype(jnp.bfloat16),
             jnp.ones((1, BLK), jnp.bfloat16),
             jnp.zeros((RHS - d_h - 1, BLK), jnp.bfloat16)], axis=0)

    h3_ref[0] = h_new


def _readout_body(h3_ref, wr1mt_ref, br1mt_ref, wr2mt_ref, br2mt_ref,
                  wr1vt_ref, br1vt_ref, wr2vt_ref, br2vt_ref,
                  zm_ref, zlv_ref):
    n = h3_ref.shape[2]
    for b in range(h3_ref.shape[0]):
        pool = (jnp.sum(h3_ref[b], axis=1, keepdims=True) * (1.0 / n))
        hm = jax.nn.relu(
            jnp.dot(wr1mt_ref[...], pool, preferred_element_type=jnp.float32)
            + br1mt_ref[...])
        zm = jnp.dot(wr2mt_ref[...], hm, preferred_element_type=jnp.float32) + br2mt_ref[...]
        zm_ref[b] = zm.reshape(-1)
        hv = jax.nn.relu(
            jnp.dot(wr1vt_ref[...], pool, preferred_element_type=jnp.float32)
            + br1vt_ref[...])
        zlv = jnp.dot(wr2vt_ref[...], hv, preferred_element_type=jnp.float32) + br2vt_ref[...]
        zlv_ref[b] = zlv.reshape(-1)


def kernel(x, adj, W_in, b_in, Wm1, bm1, Wm2, bm2, Wm3, bm3,
           Wr1m, br1m, Wr2m, br2m, Wr1v, br1v, Wr2v, br2v):
    B, N, D_IN = x.shape
    D_H = W_in.shape[1]
    D_Z = Wr2m.shape[1]
    nb = N // BLK

    WinT = W_in.T
    Wm1aT, Wm1bT = Wm1[:D_H].T, Wm1[D_H:].T
    Wm2T, Wm3T = Wm2.T, Wm3.T
    Wr1mT, Wr2mT, Wr1vT, Wr2vT = Wr1m.T, Wr2m.T, Wr1v.T, Wr2v.T
    col = lambda v: v.reshape(-1, 1)
    binT, bm1T, bm2T, bm3T = col(b_in), col(bm1), col(bm2), col(bm3)
    br1mT, br2mT, br1vT, br2vT = col(br1m), col(br2m), col(br1v), col(br2v)

    f32 = jnp.float32
    bf16 = jnp.bfloat16

    h0, hx0 = pl.pallas_call(
        _proj_body,
        out_shape=[jax.ShapeDtypeStruct((B, D_H, N), f32),
                   jax.ShapeDtypeStruct((B, RHS, N), bf16)],
    )(x, WinT, binT)

    def full(arr):
        return pl.BlockSpec(arr.shape, lambda t, b, i: (0,) * arr.ndim)

    def h3_idx(t, b, i):
        return (b, 0, i)

    weights = (Wm1aT, Wm1bT, bm1T, Wm2T, bm2T, Wm3T, bm3T)

    h3 = pl.pallas_call(
        _main_body,
        grid=(3, B, nb),
        in_specs=[pl.BlockSpec((1, BLK, N), lambda t, b, i: (b, i, 0)),
                  full(h0), full(hx0)] + [full(w) for w in weights],
        out_specs=pl.BlockSpec((1, D_H, BLK), h3_idx),
        out_shape=jax.ShapeDtypeStruct((B, D_H, N), f32),
        scratch_shapes=[
            pltpu.VMEM((2, B, D_H, N), f32),
            pltpu.VMEM((2, B, RHS, N), bf16),
            pltpu.SemaphoreType.DMA,
            pltpu.SemaphoreType.DMA,
        ],
        compiler_params=pltpu.CompilerParams(
            dimension_semantics=("arbitrary", "arbitrary", "arbitrary")),
    )(adj, h0, hx0, *weights)

    zm, zlv = pl.pallas_call(
        _readout_body,
        out_shape=[jax.ShapeDtypeStruct((B, D_Z), f32),
                   jax.ShapeDtypeStruct((B, D_Z), f32)],
    )(h3, Wr1mT, br1mT, Wr2mT, br2mT, Wr1vT, br1vT, Wr2vT, br2vT)
    return (zm, zlv)
